# baseline (speedup 1.0000x reference)
import functools

import jax
import jax.numpy as jnp
from jax import lax
from jax.experimental import pallas as pl
from jax.experimental.pallas import tpu as pltpu
from jax.experimental.pallas import tpu_sc as plsc

_N = 50000
_E = 800000
_LN_EPS = 1e-5

_F32 = jnp.float32


def _lrelu(x):
    return jnp.where(x > 0, x, x * 0.01)


def _kron8(w):
    return jnp.kron(jnp.eye(8, dtype=w.dtype), w)


def _t8(v):
    return jnp.tile(v, 8)[None, :]


def _mean_mat():
    return jnp.kron(jnp.eye(8, dtype=_F32), jnp.full((16, 16), 1.0 / 16.0, _F32))


def _full_spec(a):
    nd = a.ndim
    return pl.BlockSpec(a.shape, lambda i, _nd=nd: (0,) * _nd)


def _mlp_ws(p):
    ws = [p["inp"]["W"], p["inp"]["b"][None, :], p["hidden"][0]["W"],
          p["hidden"][0]["b"][None, :], p["out"]["W"], p["out"]["b"][None, :]]
    if "ln" in p:
        ws += [p["ln"]["g"][None, :], p["ln"]["b"][None, :]]
    return ws


def _mlp_apply(x, w, has_ln):
    h = _lrelu(jnp.dot(x, w[0][...], preferred_element_type=_F32) + w[1][...])
    h = _lrelu(jnp.dot(h, w[2][...], preferred_element_type=_F32) + w[3][...])
    f = jnp.dot(h, w[4][...], preferred_element_type=_F32) + w[5][...]
    if has_ln:
        mu = jnp.mean(f, axis=-1, keepdims=True)
        var = jnp.mean((f - mu) * (f - mu), axis=-1, keepdims=True)
        f = (f - mu) * lax.rsqrt(var + _LN_EPS) * w[6][...] + w[7][...]
    return f


def _edge_proc(pe, gs, gd, p):
    e = pe.shape[0]
    be = 8000
    ws = _mlp_ws(p)

    def body(per, gsr, gdr, *a):
        o = a[-1]
        x = per[...]
        cat = jnp.concatenate([x, gsr[...], gdr[...]], axis=1)
        o[...] = _mlp_apply(cat, a[:-1], True) + x

    blk = pl.BlockSpec((be, 16), lambda i: (i, 0))
    return pl.pallas_call(
        body,
        grid=(e // be,),
        in_specs=[blk, blk, blk] + [_full_spec(a) for a in ws],
        out_specs=blk,
        out_shape=jax.ShapeDtypeStruct((e, 16), _F32),
    )(pe, gs, gd, *ws)


def _node_proc(pn, agg2, p):
    n = pn.shape[0]
    bn = 10000
    ws = _mlp_ws(p)

    def body(pnr, agr, *a):
        o = a[-1]
        x = pnr[...]
        cat = jnp.concatenate([x, agr[0] + agr[1]], axis=1)
        o[...] = _mlp_apply(cat, a[:-1], True) + x

    blk = pl.BlockSpec((bn, 16), lambda i: (i, 0))
    blk2 = pl.BlockSpec((2, bn, 16), lambda i: (0, i, 0))
    return pl.pallas_call(
        body,
        grid=(n // bn,),
        in_specs=[blk, blk2] + [_full_spec(a) for a in ws],
        out_specs=blk,
        out_shape=jax.ShapeDtypeStruct((n, 16), _F32),
    )(pn, agg2, *ws)


def _node_encoder(x, p):
    n = x.shape[0]
    bn = 10000
    ws = _mlp_ws(p)

    def body(xr, *a):
        a[-1][...] = _mlp_apply(xr[...], a[:-1], True)

    return pl.pallas_call(
        body,
        grid=(n // bn,),
        in_specs=[pl.BlockSpec((bn, 128), lambda i: (i, 0))]
        + [_full_spec(w) for w in ws],
        out_specs=pl.BlockSpec((bn, 16), lambda i: (i, 0)),
        out_shape=jax.ShapeDtypeStruct((n, 16), _F32),
    )(x, *ws)


def _edge_encoder(ef, p_red, p_rec):
    e = ef.shape[0]
    be = 8000
    wsa = _mlp_ws(p_red)
    wsb = _mlp_ws(p_rec)

    def body(efr, *a):
        o1, o2 = a[-2], a[-1]
        x = efr[...]
        o1[...] = _mlp_apply(x, a[:8], True)
        o2[...] = _mlp_apply(x, a[8:16], True)

    blk = pl.BlockSpec((be, 16), lambda i: (i, 0))
    return pl.pallas_call(
        body,
        grid=(e // be,),
        in_specs=[blk] + [_full_spec(w) for w in wsa + wsb],
        out_specs=[blk, blk],
        out_shape=[jax.ShapeDtypeStruct((e, 16), _F32),
                   jax.ShapeDtypeStruct((e, 16), _F32)],
    )(ef, *wsa, *wsb)


def _mid_decoder_encoder(pn, p_dec, p_enc):
    n = pn.shape[0]
    wsa = _mlp_ws(p_dec)
    wsb = _mlp_ws(p_enc)

    def body(pnr, *a):
        o = a[-1]
        h = _mlp_apply(pnr[...], a[:6], False)
        o[...] = _mlp_apply(h, a[6:14], True)

    bn = 10000
    blk = pl.BlockSpec((bn, 16), lambda i: (i, 0))
    return pl.pallas_call(
        body,
        grid=(n // bn,),
        in_specs=[blk] + [_full_spec(w) for w in wsa + wsb],
        out_specs=blk,
        out_shape=jax.ShapeDtypeStruct((n, 16), _F32),
    )(pn, *wsa, *wsb)


def _final_decoder(pn, p):
    n = pn.shape[0]
    ws = _mlp_ws(p)

    def body(pnr, *a):
        a[-1][...] = _mlp_apply(pnr[...], a[:-1], False)

    bn = 10000
    return pl.pallas_call(
        body,
        grid=(n // bn,),
        in_specs=[pl.BlockSpec((bn, 16), lambda i: (i, 0))]
        + [_full_spec(w) for w in ws],
        out_specs=pl.BlockSpec((bn, 2), lambda i: (i, 0)),
        out_shape=jax.ShapeDtypeStruct((n, 2), _F32),
    )(pn, *ws)


@functools.cache
def _sc_mesh():
    return plsc.VectorSubcoreMesh(core_axis_name="c", subcore_axis_name="s")


_NTILES = 32
_EPT = _E // _NTILES
_GCH = 1000
_NCH = _EPT // _GCH


def _sc_gather(pn, src, dst):
    def body(pn_h, src_h, dst_h, gs_h, gd_h, idx_s, idx_d, buf_s, buf_d,
             sem_g, sem_w):
        wid = lax.axis_index("s") * 2 + lax.axis_index("c")
        base = pl.multiple_of(wid * _EPT, 8)
        pltpu.sync_copy(src_h.at[pl.ds(base, _EPT)], idx_s)
        pltpu.sync_copy(dst_h.at[pl.ds(base, _EPT)], idx_d)
        writes = []
        for i in range(_NCH):
            b = (i % 2) * _GCH
            if i >= 2:
                for d in writes[i - 2]:
                    d.wait()
            ca = pltpu.async_copy(pn_h.at[idx_s.at[pl.ds(i * _GCH, _GCH)]],
                                  buf_s.at[pl.ds(b, _GCH)], sem_g)
            cb = pltpu.async_copy(pn_h.at[idx_d.at[pl.ds(i * _GCH, _GCH)]],
                                  buf_d.at[pl.ds(b, _GCH)], sem_g)
            ca.wait()
            cb.wait()
            w1 = pltpu.async_copy(buf_s.at[pl.ds(b, _GCH)],
                                  gs_h.at[pl.ds(base + i * _GCH, _GCH)], sem_w)
            w2 = pltpu.async_copy(buf_d.at[pl.ds(b, _GCH)],
                                  gd_h.at[pl.ds(base + i * _GCH, _GCH)], sem_w)
            writes.append((w1, w2))
        for pair in writes[-2:]:
            for d in pair:
                d.wait()

    f = pl.kernel(
        body,
        out_type=(jax.ShapeDtypeStruct((_E, 16), _F32),
                  jax.ShapeDtypeStruct((_E, 16), _F32)),
        mesh=_sc_mesh(),
        compiler_params=pltpu.CompilerParams(use_tc_tiling_on_sc=False),
        scratch_types=[
            pltpu.VMEM((_EPT,), jnp.int32),
            pltpu.VMEM((_EPT,), jnp.int32),
            pltpu.VMEM((2 * _GCH, 16), _F32),
            pltpu.VMEM((2 * _GCH, 16), _F32),
            pltpu.SemaphoreType.DMA,
            pltpu.SemaphoreType.DMA,
        ],
    )
    return f(pn, src, dst)


_SRPD = 125
_SROW = _E // _SRPD
_SRPT = _SROW // _NTILES
_NPT = _N // 16


def _sc_scatter(pe, dst2, zeros):
    def body(pe_h, dst_h, z_h, agg_h, idx2, buf, table, sem_s, sem_l):
        cid = lax.axis_index("c")
        sid = lax.axis_index("s")
        wid = sid * 2 + cid
        pltpu.sync_copy(z_h.at[pl.ds(sid * _NPT, _NPT)],
                        table.at[pl.ds(sid * _NPT, _NPT)])
        plsc.subcore_barrier()
        pltpu.sync_copy(dst_h.at[pl.ds(wid * _SRPT, _SRPT)], idx2)
        loads = [pltpu.async_copy(pe_h.at[pl.ds(wid * _EPT, _GCH)],
                                  buf.at[pl.ds(0, _GCH)], sem_l)]
        for i in range(_NCH):
            b = (i % 2) * _GCH
            loads[i].wait()
            if i + 1 < _NCH:
                nb = ((i + 1) % 2) * _GCH
                loads.append(pltpu.async_copy(
                    pe_h.at[pl.ds(wid * _EPT + (i + 1) * _GCH, _GCH)],
                    buf.at[pl.ds(nb, _GCH)], sem_l))
            descs = []
            for g in range(8):
                d = pltpu.async_copy(buf.at[pl.ds(b + g * _SRPD, _SRPD)],
                                     table.at[idx2.at[i * 8 + g]], sem_s,
                                     add=True)
                descs.append(d)
            for d in descs:
                d.wait()
        plsc.subcore_barrier()
        pltpu.sync_copy(table.at[pl.ds(sid * _NPT, _NPT)],
                        agg_h.at[cid, pl.ds(sid * _NPT, _NPT)])

    f = pl.kernel(
        body,
        out_type=jax.ShapeDtypeStruct((2, _N, 16), _F32),
        mesh=_sc_mesh(),
        compiler_params=pltpu.CompilerParams(use_tc_tiling_on_sc=False),
        scratch_types=[
            pltpu.VMEM((_SRPT, _SRPD), jnp.int32),
            pltpu.VMEM((2 * _GCH, 16), _F32),
            pltpu.VMEM_SHARED((_N, 16), _F32),
            pltpu.SemaphoreType.DMA,
            pltpu.SemaphoreType.DMA,
        ],
    )
    return f(pe, dst2, zeros)


def kernel(current_state, edge_index, efeatures, params):
    src = edge_index[0].astype(jnp.int32)
    dst = edge_index[1].astype(jnp.int32)
    dst2 = dst.reshape(_SROW, _SRPD)
    zeros = jnp.zeros((_N, 16), _F32)

    pe_red, pe_rec = _edge_encoder(efeatures, params["enc_e_red"],
                                   params["enc_e_rec"])
    pn = _node_encoder(current_state, params["enc_n_red"])

    def mp_pass(pn, pe, e_params, n_params):
        for i in range(3):
            gs, gd = _sc_gather(pn, src, dst)
            pe = _edge_proc(pe, gs, gd, e_params[i])
            agg2 = _sc_scatter(pe, dst2, zeros)
            pn = _node_proc(pn, agg2, n_params[i])
        return pn

    pn = mp_pass(pn, pe_red, params["proc_e_red"], params["proc_n_red"])
    pn = _mid_decoder_encoder(pn, params["dec_n_red"], params["enc_n_rec"])
    pn = mp_pass(pn, pe_rec, params["proc_e_rec"], params["proc_n_rec"])
    return _final_decoder(pn, params["dec_n_rec"])
